# baseline (speedup 1.0000x reference)
import functools

import jax
import jax.numpy as jnp
from jax import lax
from jax.experimental import pallas as pl
from jax.experimental.pallas import tpu as pltpu
from jax.experimental.pallas import tpu_sc as plsc

_NC = 2
_NS = 16


def _shuffle(img, inds, *, C, H, W, npairs):
    HW = H * W
    cpc = C // _NC
    ppt = npairs // _NS
    hpt = ppt // 2
    rpt = H // _NS
    slw = HW // _NS

    mesh = plsc.VectorSubcoreMesh(core_axis_name="c", subcore_axis_name="s")

    @functools.partial(
        pl.kernel,
        out_type=jax.ShapeDtypeStruct((C, H, W), jnp.float32),
        mesh=mesh,
        scratch_types=[
            pltpu.VMEM((ppt,), jnp.int32),
            pltpu.VMEM((ppt,), jnp.int32),
            pltpu.VMEM((ppt,), jnp.int32),
            pltpu.VMEM((ppt,), jnp.int32),
            pltpu.VMEM((ppt,), jnp.float32),
            pltpu.VMEM((ppt,), jnp.float32),
            pltpu.VMEM((rpt, W), jnp.float32),
            pltpu.VMEM((slw,), jnp.float32),
            pltpu.VMEM((slw,), jnp.float32),
            pltpu.VMEM((rpt, W), jnp.float32),
            pltpu.VMEM_SHARED((HW,), jnp.float32),
            pltpu.VMEM_SHARED((HW,), jnp.float32),
            pltpu.SemaphoreType.DMA,
            pltpu.SemaphoreType.DMA,
            pltpu.SemaphoreType.DMA,
            pltpu.SemaphoreType.DMA,
            pltpu.SemaphoreType.DMA,
            pltpu.SemaphoreType.DMA,
            pltpu.SemaphoreType.DMA,
        ],
    )
    def run(img_hbm, inds_hbm, out_hbm, g0i, g1i, s0i, s1i, v0, v1,
            v2d_s, v1d_s, v1d_w, v2d_w, rowA, rowB,
            ssg, sst, srw, swb, sg0, sg1, ss):
        sc = lax.axis_index("c")
        t = lax.axis_index("s")
        ch0 = sc * cpc
        band = pl.ds(t * rpt, rpt)
        sl = pl.ds(t * slw, slw)
        for gi, si, c in ((g0i, s0i, 0), (g1i, s1i, 1)):
            pltpu.sync_copy(inds_hbm.at[pl.ds(npairs + t * ppt + c * hpt, hpt)],
                            gi.at[pl.ds(0, hpt)])
            pltpu.sync_copy(inds_hbm.at[pl.ds(t * ppt + c * hpt, hpt)],
                            gi.at[pl.ds(hpt, hpt)])
            pltpu.sync_copy(inds_hbm.at[pl.ds(t * ppt + c * hpt, hpt)],
                            si.at[pl.ds(0, hpt)])
            pltpu.sync_copy(inds_hbm.at[pl.ds(npairs + t * ppt + c * hpt, hpt)],
                            si.at[pl.ds(hpt, hpt)])

        nW = W // 16

        def reshape_in(r, carry):
            for u in range(nW):
                v1d_s[pl.ds(r * W + u * 16, 16)] = v2d_s[r, pl.ds(u * 16, 16)]
            return carry

        def reshape_out(r, carry):
            for u in range(nW):
                v2d_w[r, pl.ds(u * 16, 16)] = v1d_w[pl.ds(r * W + u * 16, 16)]
            return carry

        def stage_start(k):
            pltpu.async_copy(img_hbm.at[ch0 + k, band, :], v2d_s, ssg)

        def stage_finish(k, row):
            pltpu.make_async_copy(img_hbm.at[ch0 + k, band, :], v2d_s,
                                  ssg).wait()
            lax.fori_loop(0, rpt, reshape_in, 0)
            pltpu.async_copy(v1d_s, row.at[sl], sst)
            pltpu.make_async_copy(v1d_s, row.at[sl], sst).wait()

        def gathers(row):
            pltpu.async_copy(row.at[g0i], v0, sg0)
            pltpu.async_copy(row.at[g1i], v1, sg1)

        def phase(k, X, Y):
            @pl.when(k + 1 < cpc)
            def _():
                stage_start(k + 1)

            pltpu.make_async_copy(X.at[g0i], v0, sg0).wait()
            pltpu.async_copy(v0, X.at[s0i], ss)
            pltpu.make_async_copy(X.at[g1i], v1, sg1).wait()
            pltpu.async_copy(v1, X.at[s1i], ss)

            @pl.when(k >= 1)
            def _():
                pltpu.make_async_copy(Y.at[sl], v1d_w, srw).wait()

            @pl.when(k + 1 < cpc)
            def _():
                stage_finish(k + 1, Y)

            pltpu.make_async_copy(v0, X.at[s0i], ss).wait()
            pltpu.make_async_copy(v1, X.at[s1i], ss).wait()
            plsc.subcore_barrier()

            @pl.when(k + 1 < cpc)
            def _():
                gathers(Y)

            @pl.when(k >= 2)
            def _():
                pltpu.make_async_copy(v2d_w, out_hbm.at[ch0 + k - 2, band, :],
                                      swb).wait()

            @pl.when(k >= 1)
            def _():
                lax.fori_loop(0, rpt, reshape_out, 0)
                pltpu.async_copy(v2d_w, out_hbm.at[ch0 + k - 1, band, :], swb)

            pltpu.async_copy(X.at[sl], v1d_w, srw)

        stage_start(0)
        stage_finish(0, rowA)
        plsc.subcore_barrier()
        gathers(rowA)

        def body(k2, carry):
            k = 2 * k2
            phase(k, rowA, rowB)
            phase(k + 1, rowB, rowA)
            return carry

        lax.fori_loop(0, cpc // 2, body, 0)
        pltpu.make_async_copy(rowB.at[sl], v1d_w, srw).wait()
        pltpu.make_async_copy(v2d_w, out_hbm.at[ch0 + cpc - 2, band, :],
                              swb).wait()
        lax.fori_loop(0, rpt, reshape_out, 0)
        pltpu.async_copy(v2d_w, out_hbm.at[ch0 + cpc - 1, band, :], swb)
        pltpu.make_async_copy(v2d_w, out_hbm.at[ch0 + cpc - 1, band, :],
                              swb).wait()

    return run(img, inds)


def kernel(img, inds):
    C, H, W = img.shape
    npairs = inds.shape[0] // 2
    return _shuffle(img, inds, C=C, H=H, W=W, npairs=npairs)


# speedup vs baseline: 1.6254x; 1.6254x over previous
import functools

import jax
import jax.numpy as jnp
from jax import lax
from jax.experimental import pallas as pl
from jax.experimental.pallas import tpu as pltpu
from jax.experimental.pallas import tpu_sc as plsc

_NC = 2
_NS = 16


def _shuffle(img, inds, *, C, H, W, npairs):
    HW = H * W
    cpc = C // _NC
    ppt = npairs // _NS
    hpt = ppt // 2
    rpt = H // _NS
    slw = HW // _NS

    mesh = plsc.VectorSubcoreMesh(core_axis_name="c", subcore_axis_name="s")

    @functools.partial(
        pl.kernel,
        out_type=jax.ShapeDtypeStruct((C, H, W), jnp.float32),
        mesh=mesh,
        scratch_types=[
            pltpu.VMEM((ppt,), jnp.int32),
            pltpu.VMEM((ppt,), jnp.int32),
            pltpu.VMEM((ppt,), jnp.int32),
            pltpu.VMEM((ppt,), jnp.int32),
            pltpu.VMEM((ppt,), jnp.float32),
            pltpu.VMEM((ppt,), jnp.float32),
            pltpu.VMEM((rpt, W), jnp.float32),
            pltpu.VMEM((slw,), jnp.float32),
            pltpu.VMEM((slw,), jnp.float32),
            pltpu.VMEM((rpt, W), jnp.float32),
            pltpu.VMEM_SHARED((HW,), jnp.float32),
            pltpu.VMEM_SHARED((HW,), jnp.float32),
            pltpu.SemaphoreType.DMA,
            pltpu.SemaphoreType.DMA,
            pltpu.SemaphoreType.DMA,
            pltpu.SemaphoreType.DMA,
            pltpu.SemaphoreType.DMA,
            pltpu.SemaphoreType.DMA,
            pltpu.SemaphoreType.DMA,
        ],
    )
    def run(img_hbm, inds_hbm, out_hbm, g0i, g1i, s0i, s1i, v0, v1,
            v2d_s, v1d_s, v1d_w, v2d_w, rowA, rowB,
            ssg, sst, srw, swb, sg0, sg1, ss):
        sc = lax.axis_index("c")
        t = lax.axis_index("s")
        ch0 = sc * cpc
        band = pl.ds(t * rpt, rpt)
        sl = pl.ds(t * slw, slw)
        for gi, si, c in ((g0i, s0i, 0), (g1i, s1i, 1)):
            pltpu.sync_copy(inds_hbm.at[pl.ds(npairs + t * ppt + c * hpt, hpt)],
                            gi.at[pl.ds(0, hpt)])
            pltpu.sync_copy(inds_hbm.at[pl.ds(t * ppt + c * hpt, hpt)],
                            gi.at[pl.ds(hpt, hpt)])
            pltpu.sync_copy(inds_hbm.at[pl.ds(t * ppt + c * hpt, hpt)],
                            si.at[pl.ds(0, hpt)])
            pltpu.sync_copy(inds_hbm.at[pl.ds(npairs + t * ppt + c * hpt, hpt)],
                            si.at[pl.ds(hpt, hpt)])

        nW = W // 16

        def reshape_in(r, carry):
            vals = [v2d_s[r, pl.ds(u * 16, 16)] for u in range(nW)]
            for u in range(nW):
                v1d_s[pl.ds(r * W + u * 16, 16)] = vals[u]
            return carry

        def reshape_out(r, carry):
            vals = [v1d_w[pl.ds(r * W + u * 16, 16)] for u in range(nW)]
            for u in range(nW):
                v2d_w[r, pl.ds(u * 16, 16)] = vals[u]
            return carry

        def stage_start(k):
            pltpu.async_copy(img_hbm.at[ch0 + k, band, :], v2d_s, ssg)

        def stage_finish(k, row):
            pltpu.make_async_copy(img_hbm.at[ch0 + k, band, :], v2d_s,
                                  ssg).wait()
            lax.fori_loop(0, rpt, reshape_in, 0)
            pltpu.async_copy(v1d_s, row.at[sl], sst)
            pltpu.make_async_copy(v1d_s, row.at[sl], sst).wait()

        def gathers(row):
            pltpu.async_copy(row.at[g0i], v0, sg0)
            pltpu.async_copy(row.at[g1i], v1, sg1)

        def phase(k, X, Y):
            @pl.when(k + 1 < cpc)
            def _():
                stage_start(k + 1)

            pltpu.make_async_copy(X.at[g0i], v0, sg0).wait()
            pltpu.async_copy(v0, X.at[s0i], ss)
            pltpu.make_async_copy(X.at[g1i], v1, sg1).wait()
            pltpu.async_copy(v1, X.at[s1i], ss)

            @pl.when(k >= 1)
            def _():
                pltpu.make_async_copy(Y.at[sl], v1d_w, srw).wait()

            @pl.when(k + 1 < cpc)
            def _():
                stage_finish(k + 1, Y)

            pltpu.make_async_copy(v0, X.at[s0i], ss).wait()
            pltpu.make_async_copy(v1, X.at[s1i], ss).wait()
            plsc.subcore_barrier()

            @pl.when(k + 1 < cpc)
            def _():
                gathers(Y)

            @pl.when(k >= 2)
            def _():
                pltpu.make_async_copy(v2d_w, out_hbm.at[ch0 + k - 2, band, :],
                                      swb).wait()

            @pl.when(k >= 1)
            def _():
                lax.fori_loop(0, rpt, reshape_out, 0)
                pltpu.async_copy(v2d_w, out_hbm.at[ch0 + k - 1, band, :], swb)

            pltpu.async_copy(X.at[sl], v1d_w, srw)

        stage_start(0)
        stage_finish(0, rowA)
        plsc.subcore_barrier()
        gathers(rowA)

        def body(k2, carry):
            k = 2 * k2
            phase(k, rowA, rowB)
            phase(k + 1, rowB, rowA)
            return carry

        lax.fori_loop(0, cpc // 2, body, 0)
        pltpu.make_async_copy(rowB.at[sl], v1d_w, srw).wait()
        pltpu.make_async_copy(v2d_w, out_hbm.at[ch0 + cpc - 2, band, :],
                              swb).wait()
        lax.fori_loop(0, rpt, reshape_out, 0)
        pltpu.async_copy(v2d_w, out_hbm.at[ch0 + cpc - 1, band, :], swb)
        pltpu.make_async_copy(v2d_w, out_hbm.at[ch0 + cpc - 1, band, :],
                              swb).wait()

    return run(img, inds)


def kernel(img, inds):
    C, H, W = img.shape
    npairs = inds.shape[0] // 2
    return _shuffle(img, inds, C=C, H=H, W=W, npairs=npairs)
